# baseline (speedup 1.0000x reference)
import functools

import jax
import jax.numpy as jnp
from jax import lax
from jax.experimental import pallas as pl
from jax.experimental.pallas import tpu as pltpu
from jax.experimental.pallas import tpu_sc as plsc

N_PARTS = 24
POSE_DIM = 10
ROW = N_PARTS * POSE_DIM
ROT_ROWS = 4 * N_PARTS
TRA_ROWS = 3 * N_PARTS
SCA_ROWS = 3 * N_PARTS

NC = 2
NS = 16
NW = NC * NS

L = 16
UNROLL = 8


@functools.lru_cache(maxsize=None)
def _make_gather(B: int, V: int):
    rows_per_w = -(-ROW // NW)
    seg = (V // 3) // 128 * 128
    segments = [(0, seg), (seg, seg), (2 * seg, V - 2 * seg)]
    buf_n = max(s for _, s in segments)
    mesh = plsc.VectorSubcoreMesh(core_axis_name="c", subcore_axis_name="s")

    @functools.partial(
        pl.kernel,
        mesh=mesh,
        out_type=(
            jax.ShapeDtypeStruct((ROT_ROWS, B), jnp.float32),
            jax.ShapeDtypeStruct((TRA_ROWS, B), jnp.float32),
            jax.ShapeDtypeStruct((SCA_ROWS, B), jnp.float32),
        ),
        scratch_types=[
            pltpu.VMEM((buf_n,), jnp.float32),
            pltpu.VMEM((buf_n,), jnp.float32),
            pltpu.VMEM((B,), jnp.int32),
            pltpu.VMEM((B,), jnp.float32),
            pltpu.VMEM((B,), jnp.float32),
            pltpu.SemaphoreType.DMA,
            pltpu.SemaphoreType.DMA,
            pltpu.SemaphoreType.DMA,
            pltpu.SemaphoreType.DMA,
        ],
        compiler_params=pltpu.CompilerParams(use_tc_tiling_on_sc=True,
                                             needs_layout_passes=False),
    )
    def lane_gather(table_hbm, idx_hbm, rot_hbm, tra_hbm, sca_hbm,
                    col_a, col_b, idx_v, out_a, out_b,
                    sem_a, sem_b, sem_oa, sem_ob):
        wid = lax.axis_index("s") * NC + lax.axis_index("c")
        last_row_ok = wid < (ROW - (rows_per_w - 1) * NW)
        pltpu.sync_copy(idx_hbm, idx_v)
        lane_iota = jax.lax.iota(jnp.int32, L)

        cols = (col_a, col_b)
        csems = (sem_a, sem_b)
        outs = (out_a, out_b)
        osems = (sem_oa, sem_ob)
        units = [(j, k) for j in range(rows_per_w) for k in range(3)]

        def row_of(j):
            return j * NW + wid

        def guard(j, body):
            if j == rows_per_w - 1:
                pl.when(last_row_ok)(body)
            else:
                body()

        def start_stream(u):
            j, k = units[u]
            base, n = segments[k]

            def _start():
                pltpu.async_copy(
                    table_hbm.at[row_of(j), pl.ds(base, n)],
                    cols[u % 2].at[pl.ds(0, n)], csems[u % 2])
            guard(j, _start)

        def wait_stream(u):
            j, k = units[u]
            base, n = segments[k]

            def _wait():
                pltpu.make_async_copy(
                    table_hbm.at[row_of(j), pl.ds(base, n)],
                    cols[u % 2].at[pl.ds(0, n)], csems[u % 2]).wait()
            guard(j, _wait)

        def gather_pass(u):
            j, k = units[u]
            base, n = segments[k]
            cbuf = cols[u % 2]
            obuf = outs[j % 2]

            def body(i, _):
                b0 = i * (UNROLL * L)
                for uu in range(UNROLL):
                    o = b0 + uu * L
                    iv = idx_v[pl.ds(o, L)]
                    if k == 0:
                        m = iv < n
                        sh = jnp.where(m, iv, 0)
                        g = plsc.load_gather(cbuf, [sh], mask=m)
                        obuf[pl.ds(o, L)] = g
                    else:
                        sh = iv - base
                        if k == 2:
                            m = sh >= 0
                        else:
                            m = (sh >= 0) & (sh < n)
                        shs = jnp.where(m, sh, 0)
                        g = plsc.load_gather(cbuf, [shs], mask=m)
                        plsc.store_scatter(obuf, [lane_iota + o], g, mask=m)
                return 0

            def _run():
                lax.fori_loop(0, B // (UNROLL * L), body, 0)

            guard(j, _run)

        def start_flush(j):
            obuf = outs[j % 2]
            r = row_of(j)
            lo, hi = j * NW, j * NW + NW
            targets = []
            if lo < ROT_ROWS and hi > 0:
                targets.append((rot_hbm, 0))
            if lo < ROT_ROWS + TRA_ROWS and hi > ROT_ROWS:
                targets.append((tra_hbm, ROT_ROWS))
            if hi > ROT_ROWS + TRA_ROWS:
                targets.append((sca_hbm, ROT_ROWS + TRA_ROWS))

            def _flush():
                for out_ref, tbase in targets:
                    nrows = out_ref.shape[0]
                    if len(targets) == 1:
                        pltpu.async_copy(obuf, out_ref.at[r - tbase],
                                         osems[j % 2])
                    else:
                        @pl.when((r >= tbase) & (r < tbase + nrows))
                        def _store():
                            pltpu.async_copy(obuf, out_ref.at[r - tbase],
                                             osems[j % 2])
            guard(j, _flush)

        def wait_flush(j):
            def _wait():
                pltpu.make_async_copy(rot_hbm.at[0], outs[j % 2],
                                      osems[j % 2]).wait()
            guard(j, _wait)

        start_stream(0)
        for u, (j, k) in enumerate(units):
            wait_stream(u)
            if u + 1 < len(units):
                start_stream(u + 1)
            if k == 0 and j >= 2:
                wait_flush(j - 2)
            gather_pass(u)
            if k == 2:
                start_flush(j)
        wait_flush(rows_per_w - 2)
        wait_flush(rows_per_w - 1)

    return lane_gather


def kernel(input, weight):
    B = input.shape[0]
    V = weight.shape[0]
    tableT = weight.transpose(2, 1, 0).reshape(ROW, V)
    rot_f, tra_f, sca_f = _make_gather(B, V)(tableT, input)
    rotation = rot_f.reshape(4, N_PARTS, B).transpose(2, 1, 0)
    translation = tra_f.reshape(3, N_PARTS, B).transpose(2, 1, 0)
    scale = sca_f.reshape(3, N_PARTS, B).transpose(2, 1, 0)
    return rotation, translation, scale


# speedup vs baseline: 2.1380x; 2.1380x over previous
import functools

import jax
import jax.numpy as jnp
from jax import lax
from jax.experimental import pallas as pl
from jax.experimental.pallas import tpu as pltpu
from jax.experimental.pallas import tpu_sc as plsc

N_PARTS = 24
POSE_DIM = 10
ROW = N_PARTS * POSE_DIM
ROT_ROWS = 4 * N_PARTS
TRA_ROWS = 3 * N_PARTS
SCA_ROWS = 3 * N_PARTS

NC = 2
NS = 16
NW = NC * NS

L = 16
UNROLL = 16
OUT_CHUNK = 4096


@functools.lru_cache(maxsize=None)
def _make_gather(B: int, V: int):
    rows_per_w = -(-ROW // NW)
    nchunk = B // OUT_CHUNK
    assert nchunk * OUT_CHUNK == B and nchunk % 2 == 0
    mesh = plsc.VectorSubcoreMesh(core_axis_name="c", subcore_axis_name="s")

    @functools.partial(
        pl.kernel,
        mesh=mesh,
        out_type=(
            jax.ShapeDtypeStruct((ROT_ROWS, B), jnp.float32),
            jax.ShapeDtypeStruct((TRA_ROWS, B), jnp.float32),
            jax.ShapeDtypeStruct((SCA_ROWS, B), jnp.float32),
        ),
        scratch_types=[
            pltpu.VMEM((V,), jnp.float32),
            pltpu.VMEM((B,), jnp.int32),
            pltpu.VMEM((OUT_CHUNK,), jnp.float32),
            pltpu.VMEM((OUT_CHUNK,), jnp.float32),
            pltpu.SemaphoreType.DMA,
            pltpu.SemaphoreType.DMA,
            pltpu.SemaphoreType.DMA,
        ],
        compiler_params=pltpu.CompilerParams(use_tc_tiling_on_sc=True,
                                             needs_layout_passes=False),
    )
    def lane_gather(table_hbm, idx_hbm, rot_hbm, tra_hbm, sca_hbm,
                    col_v, idx_v, out_a, out_b, csem, sem_oa, sem_ob):
        wid = lax.axis_index("s") * NC + lax.axis_index("c")
        last_row_ok = wid < (ROW - (rows_per_w - 1) * NW)
        outs = (out_a, out_b)
        osems = (sem_oa, sem_ob)

        def row_of(j):
            return j * NW + wid

        def guard(j, body):
            if j == rows_per_w - 1:
                pl.when(last_row_ok)(body)
            else:
                body()

        def start_stream(j):
            def _start():
                pltpu.async_copy(table_hbm.at[row_of(j)], col_v, csem)
            guard(j, _start)

        def wait_stream(j):
            def _wait():
                pltpu.make_async_copy(table_hbm.at[row_of(j)], col_v,
                                      csem).wait()
            guard(j, _wait)

        def targets_for(j):
            lo, hi = j * NW, j * NW + NW
            targets = []
            if lo < ROT_ROWS and hi > 0:
                targets.append((rot_hbm, 0))
            if lo < ROT_ROWS + TRA_ROWS and hi > ROT_ROWS:
                targets.append((tra_hbm, ROT_ROWS))
            if hi > ROT_ROWS + TRA_ROWS:
                targets.append((sca_hbm, ROT_ROWS + TRA_ROWS))
            return targets

        def process_row(j):
            r = row_of(j)
            targets = targets_for(j)

            def _run():
                for jc in range(nchunk):
                    obuf = outs[jc % 2]
                    osem = osems[jc % 2]
                    if j > 0 or jc >= 2:
                        pltpu.make_async_copy(
                            rot_hbm.at[0, pl.ds(0, OUT_CHUNK)], obuf,
                            osem).wait()

                    def body(i, _):
                        b0 = i * (UNROLL * L)
                        for u in range(UNROLL):
                            o = b0 + u * L
                            g = plsc.load_gather(
                                col_v,
                                [idx_v[pl.ds(jc * OUT_CHUNK + o, L)]])
                            obuf[pl.ds(o, L)] = g
                        return 0
                    lax.fori_loop(0, OUT_CHUNK // (UNROLL * L), body, 0)

                    for out_ref, tbase in targets:
                        nrows = out_ref.shape[0]

                        def _store(out_ref=out_ref, tbase=tbase):
                            pltpu.async_copy(
                                obuf,
                                out_ref.at[r - tbase,
                                           pl.ds(jc * OUT_CHUNK, OUT_CHUNK)],
                                osem)
                        if len(targets) == 1:
                            _store()
                        else:
                            pl.when((r >= tbase) & (r < tbase + nrows))(
                                _store)
            guard(j, _run)

        start_stream(0)
        pltpu.sync_copy(idx_hbm, idx_v)
        for j in range(rows_per_w):
            wait_stream(j)
            process_row(j)
            if j + 1 < rows_per_w:
                start_stream(j + 1)
        for p in range(2):
            pltpu.make_async_copy(rot_hbm.at[0, pl.ds(0, OUT_CHUNK)],
                                  outs[p], osems[p]).wait()

    return lane_gather


def kernel(input, weight):
    B = input.shape[0]
    V = weight.shape[0]
    tableT = weight.transpose(2, 1, 0).reshape(ROW, V)
    rot_f, tra_f, sca_f = _make_gather(B, V)(tableT, input)
    rotation = rot_f.reshape(4, N_PARTS, B).transpose(2, 1, 0)
    translation = tra_f.reshape(3, N_PARTS, B).transpose(2, 1, 0)
    scale = sca_f.reshape(3, N_PARTS, B).transpose(2, 1, 0)
    return rotation, translation, scale


# speedup vs baseline: 2.7444x; 1.2836x over previous
import functools

import jax
import jax.numpy as jnp
from jax import lax
from jax.experimental import pallas as pl
from jax.experimental.pallas import tpu as pltpu
from jax.experimental.pallas import tpu_sc as plsc

N_PARTS = 24
POSE_DIM = 10
ROW = N_PARTS * POSE_DIM
ROT_ROWS = 4 * N_PARTS
TRA_ROWS = 3 * N_PARTS
SCA_ROWS = 3 * N_PARTS

NC = 2
NS = 16
NW = NC * NS

L = 16
UNROLL = 16
OUT_CHUNK = 4096


@functools.lru_cache(maxsize=None)
def _make_gather(B: int, V: int):
    rows_per_w = -(-ROW // NW)
    nchunk = B // OUT_CHUNK
    assert nchunk * OUT_CHUNK == B and nchunk % 2 == 0
    mesh = plsc.VectorSubcoreMesh(core_axis_name="c", subcore_axis_name="s")

    @functools.partial(
        pl.kernel,
        mesh=mesh,
        out_type=(
            jax.ShapeDtypeStruct((ROT_ROWS, B), jnp.float32),
            jax.ShapeDtypeStruct((TRA_ROWS, B), jnp.float32),
            jax.ShapeDtypeStruct((SCA_ROWS, B), jnp.float32),
        ),
        scratch_types=[
            pltpu.VMEM((V,), jnp.float32),
            pltpu.VMEM((B,), jnp.int32),
            pltpu.VMEM((OUT_CHUNK,), jnp.float32),
            pltpu.VMEM((OUT_CHUNK,), jnp.float32),
            pltpu.SemaphoreType.DMA,
            pltpu.SemaphoreType.DMA,
            pltpu.SemaphoreType.DMA,
        ],
        compiler_params=pltpu.CompilerParams(use_tc_tiling_on_sc=True,
                                             needs_layout_passes=False),
    )
    def lane_gather(table_hbm, idx_hbm, rot_hbm, tra_hbm, sca_hbm,
                    col_v, idx_v, out_a, out_b, csem, sem_oa, sem_ob):
        wid = lax.axis_index("s") * NC + lax.axis_index("c")
        last_row_ok = wid < (ROW - (rows_per_w - 1) * NW)
        outs = (out_a, out_b)
        osems = (sem_oa, sem_ob)

        def row_of(j):
            return j * NW + wid

        def guard(j, body):
            if j == rows_per_w - 1:
                pl.when(last_row_ok)(body)
            else:
                body()

        def start_stream(j):
            def _start():
                pltpu.async_copy(table_hbm.at[row_of(j)], col_v, csem)
            guard(j, _start)

        def wait_stream(j):
            def _wait():
                pltpu.make_async_copy(table_hbm.at[row_of(j)], col_v,
                                      csem).wait()
            guard(j, _wait)

        def targets_for(j):
            lo, hi = j * NW, j * NW + NW
            targets = []
            if lo < ROT_ROWS and hi > 0:
                targets.append((rot_hbm, 0))
            if lo < ROT_ROWS + TRA_ROWS and hi > ROT_ROWS:
                targets.append((tra_hbm, ROT_ROWS))
            if hi > ROT_ROWS + TRA_ROWS:
                targets.append((sca_hbm, ROT_ROWS + TRA_ROWS))
            return targets

        def process_row(j):
            r = row_of(j)
            targets = targets_for(j)

            def _run():
                for jc in range(nchunk):
                    obuf = outs[jc % 2]
                    osem = osems[jc % 2]
                    if j > 0 or jc >= 2:
                        pltpu.make_async_copy(
                            rot_hbm.at[0, pl.ds(0, OUT_CHUNK)], obuf,
                            osem).wait()

                    @plsc.parallel_loop(0, OUT_CHUNK // L, 1, unroll=UNROLL)
                    def _gather_loop(i):
                        o = i * L
                        g = plsc.load_gather(
                            col_v,
                            [idx_v[pl.ds(jc * OUT_CHUNK + o, L)]])
                        obuf[pl.ds(o, L)] = g

                    for out_ref, tbase in targets:
                        nrows = out_ref.shape[0]

                        def _store(out_ref=out_ref, tbase=tbase):
                            pltpu.async_copy(
                                obuf,
                                out_ref.at[r - tbase,
                                           pl.ds(jc * OUT_CHUNK, OUT_CHUNK)],
                                osem)
                        if len(targets) == 1:
                            _store()
                        else:
                            pl.when((r >= tbase) & (r < tbase + nrows))(
                                _store)
            guard(j, _run)

        start_stream(0)
        pltpu.sync_copy(idx_hbm, idx_v)
        for j in range(rows_per_w):
            wait_stream(j)
            process_row(j)
            if j + 1 < rows_per_w:
                start_stream(j + 1)
        for p in range(2):
            pltpu.make_async_copy(rot_hbm.at[0, pl.ds(0, OUT_CHUNK)],
                                  outs[p], osems[p]).wait()

    return lane_gather


def kernel(input, weight):
    B = input.shape[0]
    V = weight.shape[0]
    tableT = weight.transpose(2, 1, 0).reshape(ROW, V)
    rot_f, tra_f, sca_f = _make_gather(B, V)(tableT, input)
    rotation = rot_f.reshape(4, N_PARTS, B).transpose(2, 1, 0)
    translation = tra_f.reshape(3, N_PARTS, B).transpose(2, 1, 0)
    scale = sca_f.reshape(3, N_PARTS, B).transpose(2, 1, 0)
    return rotation, translation, scale


# speedup vs baseline: 2.9075x; 1.0594x over previous
import functools

import jax
import jax.numpy as jnp
from jax import lax
from jax.experimental import pallas as pl
from jax.experimental.pallas import tpu as pltpu
from jax.experimental.pallas import tpu_sc as plsc

N_PARTS = 24
POSE_DIM = 10
ROW = N_PARTS * POSE_DIM
ROT_ROWS = 4 * N_PARTS
TRA_ROWS = 3 * N_PARTS
SCA_ROWS = 3 * N_PARTS

NC = 2
NS = 16
NW = NC * NS

L = 16
UNROLL = 16
OUT_CHUNK = 4096


@functools.lru_cache(maxsize=None)
def _make_gather(B: int, V: int):
    rows_per_w = -(-ROW // NW)
    nchunk = B // OUT_CHUNK
    assert nchunk * OUT_CHUNK == B and nchunk % 2 == 0
    mesh = plsc.VectorSubcoreMesh(core_axis_name="c", subcore_axis_name="s")

    @functools.partial(
        pl.kernel,
        mesh=mesh,
        out_type=(
            jax.ShapeDtypeStruct((N_PARTS, B // 128, 4, 128), jnp.float32),
            jax.ShapeDtypeStruct((TRA_ROWS, B), jnp.float32),
            jax.ShapeDtypeStruct((SCA_ROWS, B), jnp.float32),
        ),
        scratch_types=[
            pltpu.VMEM((V,), jnp.float32),
            pltpu.VMEM((B,), jnp.int32),
            pltpu.VMEM((OUT_CHUNK,), jnp.float32),
            pltpu.VMEM((OUT_CHUNK,), jnp.float32),
            pltpu.SemaphoreType.DMA,
            pltpu.SemaphoreType.DMA,
            pltpu.SemaphoreType.DMA,
        ],
        compiler_params=pltpu.CompilerParams(use_tc_tiling_on_sc=True,
                                             needs_layout_passes=False),
    )
    def lane_gather(table_hbm, idx_hbm, rot_hbm, tra_hbm, sca_hbm,
                    col_v, idx_v, out_a, out_b, csem, sem_oa, sem_ob):
        wid = lax.axis_index("s") * NC + lax.axis_index("c")
        last_row_ok = wid < (ROW - (rows_per_w - 1) * NW)
        outs = (out_a, out_b)
        osems = (sem_oa, sem_ob)

        def row_of(j):
            return j * NW + wid

        def guard(j, body):
            if j == rows_per_w - 1:
                pl.when(last_row_ok)(body)
            else:
                body()

        def start_stream(j):
            def _start():
                pltpu.async_copy(table_hbm.at[row_of(j)], col_v, csem)
            guard(j, _start)

        def wait_stream(j):
            def _wait():
                pltpu.make_async_copy(table_hbm.at[row_of(j)], col_v,
                                      csem).wait()
            guard(j, _wait)

        def targets_for(j):
            lo, hi = j * NW, j * NW + NW
            targets = []
            if lo < ROT_ROWS + TRA_ROWS and hi > ROT_ROWS:
                targets.append((tra_hbm, ROT_ROWS))
            if hi > ROT_ROWS + TRA_ROWS:
                targets.append((sca_hbm, ROT_ROWS + TRA_ROWS))
            return targets

        def process_row(j):
            r = row_of(j)
            targets = targets_for(j)
            is_rot = (j + 1) * NW <= ROT_ROWS
            rc = r // N_PARTS
            rp = r % N_PARTS

            def _run():
                for jc in range(nchunk):
                    obuf = outs[jc % 2]
                    osem = osems[jc % 2]
                    if j > 0 or jc >= 2:
                        pltpu.make_async_copy(
                            tra_hbm.at[0, pl.ds(0, OUT_CHUNK)], obuf,
                            osem).wait()

                    @plsc.parallel_loop(0, OUT_CHUNK // L, 1, unroll=UNROLL)
                    def _gather_loop(i):
                        o = i * L
                        g = plsc.load_gather(
                            col_v,
                            [idx_v[pl.ds(jc * OUT_CHUNK + o, L)]])
                        obuf[pl.ds(o, L)] = g

                    if is_rot:
                        for k in range(OUT_CHUNK // 128):
                            pltpu.async_copy(
                                obuf.at[pl.ds(k * 128, 128)],
                                rot_hbm.at[rp, jc * (OUT_CHUNK // 128) + k,
                                           rc],
                                osem)
                    else:
                        for out_ref, tbase in targets:
                            nrows = out_ref.shape[0]

                            def _store(out_ref=out_ref, tbase=tbase):
                                pltpu.async_copy(
                                    obuf,
                                    out_ref.at[r - tbase,
                                               pl.ds(jc * OUT_CHUNK,
                                                     OUT_CHUNK)],
                                    osem)
                            if len(targets) == 1:
                                _store()
                            else:
                                pl.when((r >= tbase) & (r < tbase + nrows))(
                                    _store)
            guard(j, _run)

        start_stream(0)
        pltpu.sync_copy(idx_hbm, idx_v)
        for j in range(rows_per_w):
            wait_stream(j)
            process_row(j)
            if j + 1 < rows_per_w:
                start_stream(j + 1)
        for p in range(2):
            pltpu.make_async_copy(tra_hbm.at[0, pl.ds(0, OUT_CHUNK)],
                                  outs[p], osems[p]).wait()

    return lane_gather


def kernel(input, weight):
    B = input.shape[0]
    V = weight.shape[0]
    tableT = weight.transpose(2, 1, 0).reshape(ROW, V)
    rot_f, tra_f, sca_f = _make_gather(B, V)(tableT, input)
    rotation = rot_f.transpose(1, 3, 0, 2).reshape(B, N_PARTS, 4)
    translation = tra_f.reshape(3, N_PARTS, B).transpose(2, 1, 0)
    scale = sca_f.reshape(3, N_PARTS, B).transpose(2, 1, 0)
    return rotation, translation, scale


# speedup vs baseline: 2.9100x; 1.0009x over previous
import functools

import jax
import jax.numpy as jnp
from jax import lax
from jax.experimental import pallas as pl
from jax.experimental.pallas import tpu as pltpu
from jax.experimental.pallas import tpu_sc as plsc

N_PARTS = 24
POSE_DIM = 10
ROW = N_PARTS * POSE_DIM
ROT_ROWS = 4 * N_PARTS
TRA_ROWS = 3 * N_PARTS
SCA_ROWS = 3 * N_PARTS

NC = 2
NS = 16
NW = NC * NS

L = 16
UNROLL = 16
OUT_CHUNK = 4096


@functools.lru_cache(maxsize=None)
def _make_gather(B: int, V: int):
    rows_per_w = -(-ROW // NW)
    nchunk = B // OUT_CHUNK
    assert nchunk * OUT_CHUNK == B and nchunk % 2 == 0
    mesh = plsc.VectorSubcoreMesh(core_axis_name="c", subcore_axis_name="s")

    @functools.partial(
        pl.kernel,
        mesh=mesh,
        out_type=(
            jax.ShapeDtypeStruct((N_PARTS, B // 128, 4, 128), jnp.float32),
            jax.ShapeDtypeStruct((TRA_ROWS, B), jnp.float32),
            jax.ShapeDtypeStruct((SCA_ROWS, B), jnp.float32),
        ),
        scratch_types=[
            pltpu.VMEM((V,), jnp.float32),
            pltpu.VMEM((B,), jnp.int32),
            pltpu.VMEM((OUT_CHUNK,), jnp.float32),
            pltpu.VMEM((OUT_CHUNK,), jnp.float32),
            pltpu.SemaphoreType.DMA,
            pltpu.SemaphoreType.DMA,
            pltpu.SemaphoreType.DMA,
        ],
        compiler_params=pltpu.CompilerParams(use_tc_tiling_on_sc=True,
                                             needs_layout_passes=False,
                                             disable_bounds_checks=True,
                                             disable_semaphore_checks=True,
                                             skip_device_barrier=True),
    )
    def lane_gather(table_hbm, idx_hbm, rot_hbm, tra_hbm, sca_hbm,
                    col_v, idx_v, out_a, out_b, csem, sem_oa, sem_ob):
        wid = lax.axis_index("s") * NC + lax.axis_index("c")
        last_row_ok = wid < (ROW - (rows_per_w - 1) * NW)
        outs = (out_a, out_b)
        osems = (sem_oa, sem_ob)

        def row_of(j):
            return j * NW + wid

        def guard(j, body):
            if j == rows_per_w - 1:
                pl.when(last_row_ok)(body)
            else:
                body()

        def start_stream(j):
            def _start():
                pltpu.async_copy(table_hbm.at[row_of(j)], col_v, csem)
            guard(j, _start)

        def wait_stream(j):
            def _wait():
                pltpu.make_async_copy(table_hbm.at[row_of(j)], col_v,
                                      csem).wait()
            guard(j, _wait)

        def targets_for(j):
            lo, hi = j * NW, j * NW + NW
            targets = []
            if lo < ROT_ROWS + TRA_ROWS and hi > ROT_ROWS:
                targets.append((tra_hbm, ROT_ROWS))
            if hi > ROT_ROWS + TRA_ROWS:
                targets.append((sca_hbm, ROT_ROWS + TRA_ROWS))
            return targets

        def process_row(j):
            r = row_of(j)
            targets = targets_for(j)
            is_rot = (j + 1) * NW <= ROT_ROWS
            rc = r // N_PARTS
            rp = r % N_PARTS

            def _run():
                for jc in range(nchunk):
                    obuf = outs[jc % 2]
                    osem = osems[jc % 2]
                    if j > 0 or jc >= 2:
                        pltpu.make_async_copy(
                            tra_hbm.at[0, pl.ds(0, OUT_CHUNK)], obuf,
                            osem).wait()

                    @plsc.parallel_loop(0, OUT_CHUNK // L, 1, unroll=UNROLL)
                    def _gather_loop(i):
                        o = i * L
                        g = plsc.load_gather(
                            col_v,
                            [idx_v[pl.ds(jc * OUT_CHUNK + o, L)]])
                        obuf[pl.ds(o, L)] = g

                    if is_rot:
                        for k in range(OUT_CHUNK // 128):
                            pltpu.async_copy(
                                obuf.at[pl.ds(k * 128, 128)],
                                rot_hbm.at[rp, jc * (OUT_CHUNK // 128) + k,
                                           rc],
                                osem)
                    else:
                        for out_ref, tbase in targets:
                            nrows = out_ref.shape[0]

                            def _store(out_ref=out_ref, tbase=tbase):
                                pltpu.async_copy(
                                    obuf,
                                    out_ref.at[r - tbase,
                                               pl.ds(jc * OUT_CHUNK,
                                                     OUT_CHUNK)],
                                    osem)
                            if len(targets) == 1:
                                _store()
                            else:
                                pl.when((r >= tbase) & (r < tbase + nrows))(
                                    _store)
            guard(j, _run)

        start_stream(0)
        pltpu.sync_copy(idx_hbm, idx_v)
        for j in range(rows_per_w):
            wait_stream(j)
            process_row(j)
            if j + 1 < rows_per_w:
                start_stream(j + 1)
        for p in range(2):
            pltpu.make_async_copy(tra_hbm.at[0, pl.ds(0, OUT_CHUNK)],
                                  outs[p], osems[p]).wait()

    return lane_gather


def kernel(input, weight):
    B = input.shape[0]
    V = weight.shape[0]
    tableT = weight.transpose(2, 1, 0).reshape(ROW, V)
    rot_f, tra_f, sca_f = _make_gather(B, V)(tableT, input)
    rotation = rot_f.transpose(1, 3, 0, 2).reshape(B, N_PARTS, 4)
    translation = tra_f.reshape(3, N_PARTS, B).transpose(2, 1, 0)
    scale = sca_f.reshape(3, N_PARTS, B).transpose(2, 1, 0)
    return rotation, translation, scale
